# Initial kernel scaffold; baseline (speedup 1.0000x reference)
import jax
import jax.numpy as jnp
from jax.experimental import pallas as pl


def kernel(e, edge_index, edge_p, bn_gamma, bn_beta, Wi, bi, Wh, bh):
    raise NotImplementedError("write your pallas kernel here")


# speedup vs baseline: 3.6638x; 3.6638x over previous
import functools

import jax
import jax.numpy as jnp
from jax import lax
from jax.experimental import pallas as pl
from jax.experimental.pallas import tpu as pltpu
from jax.experimental.pallas import tpu_sc as plsc

N = 100000
E = 1600000
IN_FEATS = 16
OUT_FEATS = 32
DEPTH = 3
EPS = 1e-5

L = 16
NTILES = 16
NP = 100096
TRASH = N
K = 256
NB = K // 128
EPT = 100352
NCH = EPT // K
EP = EPT * NTILES
ERB = EP // 128
ART = NP // NTILES
CN = 102400
CNT_ = CN // NTILES
ZROWS = 368
ZREP = ART // ZROWS
CZN = 1600

_f32 = jnp.float32
_i32 = jnp.int32


def _mesh():
    return plsc.VectorSubcoreMesh(core_axis_name="c", subcore_axis_name="s")


def _zero_rows(ref, n):
    def body(i, _):
        ref[i, :] = jnp.zeros((L,), _f32)
        return 0
    lax.fori_loop(0, n, body, 0, unroll=4)


def _sc_scatter_feats(dstoff, f2):
    @functools.partial(
        pl.kernel,
        out_type=(
            jax.ShapeDtypeStruct((2 * NP, L), _f32),
            jax.ShapeDtypeStruct((2 * CN,), _f32),
        ),
        mesh=_mesh(),
        compiler_params=pltpu.CompilerParams(use_tc_tiling_on_sc=False),
        scratch_types=[
            pltpu.VMEM_SHARED((NP, L), _f32),
            pltpu.VMEM_SHARED((CN,), _f32),
            pltpu.VMEM((ZROWS, L), _f32),
            pltpu.VMEM((CZN,), _f32),
            pltpu.VMEM((128,), _f32),
            pltpu.VMEM((NB, 128), _i32),
            pltpu.VMEM((K, L), _f32),
            pltpu.SemaphoreType.DMA,
        ],
    )
    def k(dst_hbm, f_hbm, s_hbm, cnt_hbm, acc, cntacc, zbuf, zbuf1, onesb,
          idxd, buff, sem):
        c = lax.axis_index("c")
        s = lax.axis_index("s")
        _zero_rows(zbuf, ZROWS)
        for j in range(ZREP):
            pltpu.sync_copy(zbuf, acc.at[pl.ds(s * ART + j * ZROWS, ZROWS)])

        def z1(i, _):
            zbuf1[pl.ds(i * L, L)] = jnp.zeros((L,), _f32)
            return 0
        lax.fori_loop(0, CZN // L, z1, 0, unroll=4)
        for i in range(128 // L):
            onesb[pl.ds(i * L, L)] = jnp.ones((L,), _f32)
        for j in range(CNT_ // CZN):
            pltpu.sync_copy(zbuf1, cntacc.at[pl.ds(s * CNT_ + j * CZN, CZN)])
        plsc.subcore_barrier()

        base_row = s * (EPT // 128)

        def chunk(jc, _):
            r0 = base_row + jc * NB
            pltpu.sync_copy(dst_hbm.at[pl.ds(r0, NB)], idxd)
            off = c * EP + s * EPT + jc * K
            cp = pltpu.async_copy(f_hbm.at[pl.ds(off, K)], buff, sem)

            half = NCH // 2

            @pl.when(jnp.logical_and(jc >= c * half, jc < (c + 1) * half))
            def _():
                for k_ in range(NB):
                    pltpu.sync_copy(onesb, cntacc.at[idxd.at[k_]], add=True)

            cp.wait()
            for k_ in range(NB):
                pltpu.sync_copy(buff.at[pl.ds(k_ * 128, 128)],
                                acc.at[idxd.at[k_]], add=True)
            return 0

        lax.fori_loop(0, NCH, chunk, 0)
        plsc.subcore_barrier()
        pltpu.sync_copy(acc.at[pl.ds(s * ART, ART)],
                        s_hbm.at[pl.ds(c * NP + s * ART, ART)])
        pltpu.sync_copy(cntacc.at[pl.ds(s * CNT_, CNT_)],
                        cnt_hbm.at[pl.ds(c * CN + s * CNT_, CNT_)])

    return k(dstoff, f2)


def _sc_layer(srcoff, dstoff, c2, a2, b2):
    @functools.partial(
        pl.kernel,
        out_type=jax.ShapeDtypeStruct((2 * NP, L), _f32),
        mesh=_mesh(),
        compiler_params=pltpu.CompilerParams(use_tc_tiling_on_sc=False),
        scratch_types=[
            pltpu.VMEM_SHARED((NP, L), _f32),
            pltpu.VMEM((ZROWS, L), _f32),
            pltpu.VMEM((NB, 128), _i32),
            pltpu.VMEM((NB, 128), _i32),
            pltpu.VMEM((NB, 128), _i32),
            pltpu.VMEM((K, L), _f32),
            pltpu.VMEM((K, L), _f32),
            pltpu.VMEM((K, L), _f32),
            pltpu.SemaphoreType.DMA,
            pltpu.SemaphoreType.DMA,
            pltpu.SemaphoreType.DMA,
        ],
    )
    def k(src_hbm, dst_hbm, c_hbm, a_hbm, b_hbm, s_hbm, acc, zbuf,
          idxs, idxdo, idxd, bufa, bufb, bufc, sema, semb, semc):
        c = lax.axis_index("c")
        s = lax.axis_index("s")
        _zero_rows(zbuf, ZROWS)
        for j in range(ZREP):
            pltpu.sync_copy(zbuf, acc.at[pl.ds(s * ART + j * ZROWS, ZROWS)])
        plsc.subcore_barrier()

        def chunk(jc, _):
            r0 = s * (EPT // 128) + jc * NB
            pltpu.sync_copy(src_hbm.at[pl.ds(c * ERB + r0, NB)], idxs)
            pltpu.sync_copy(dst_hbm.at[pl.ds(c * ERB + r0, NB)], idxdo)
            pltpu.sync_copy(dst_hbm.at[pl.ds(r0, NB)], idxd)
            off = c * EP + s * EPT + jc * K
            cps = [pltpu.async_copy(c_hbm.at[pl.ds(off, K)], bufc, semc)]
            for k_ in range(NB):
                cps.append(pltpu.async_copy(
                    a_hbm.at[idxs.at[k_]],
                    bufa.at[pl.ds(k_ * 128, 128)], sema))
            for k_ in range(NB):
                cps.append(pltpu.async_copy(
                    b_hbm.at[idxdo.at[k_]],
                    bufb.at[pl.ds(k_ * 128, 128)], semb))
            for cp in cps:
                cp.wait()

            def compute(i, _):
                y = bufa[i, :] + bufb[i, :] + bufc[i, :]
                bufc[i, :] = jnp.maximum(y, 0.0)
                return 0

            lax.fori_loop(0, K, compute, 0, unroll=8)
            for k_ in range(NB):
                pltpu.sync_copy(bufc.at[pl.ds(k_ * 128, 128)],
                                acc.at[idxd.at[k_]], add=True)
            return 0

        lax.fori_loop(0, NCH, chunk, 0)
        plsc.subcore_barrier()
        pltpu.sync_copy(acc.at[pl.ds(s * ART, ART)],
                        s_hbm.at[pl.ds(c * NP + s * ART, ART)])

    return k(srcoff, dstoff, c2, a2, b2)


def _planes(x):
    return jnp.concatenate([x[:, :L], x[:, L:]], axis=0)


def _unplanes(sp):
    return jnp.concatenate([sp[:NP], sp[NP:]], axis=1)


def kernel(e, edge_index, edge_p, bn_gamma, bn_beta, Wi, bi, Wh, bh):
    src = edge_index[0]
    dst = edge_index[1]

    mean = jnp.mean(e, axis=0)
    var = jnp.var(e, axis=0)
    scale = bn_gamma / jnp.sqrt(var + EPS)
    shift = bn_beta - mean * scale
    W1 = Wi * scale[None, :]
    b1 = Wi @ shift + bi

    Ws = Wh[:, :OUT_FEATS]
    Wf = Wh[:, OUT_FEATS:2 * OUT_FEATS]
    wp = Wh[:, 2 * OUT_FEATS]
    Wd = Wh[:, 2 * OUT_FEATS + 1:]

    f_e = jax.nn.relu(e @ W1.T + b1)
    cmat = f_e @ Wf.T + edge_p * wp[None, :] + bh

    pad = EP - E
    f_p = jnp.pad(f_e, ((0, pad), (0, 0)))
    c_p = jnp.pad(cmat, ((0, pad), (0, 0)))
    src_p = jnp.pad(src, (0, pad))
    dst_p = jnp.pad(dst, (0, pad), constant_values=TRASH)

    offs = jnp.array([0, NP], _i32)
    srcoff = (src_p[None, :] + offs[:, None]).reshape(2 * ERB, 128)
    dstoff = (dst_p[None, :] + offs[:, None]).reshape(2 * ERB, 128)

    f2 = _planes(f_p)
    c2 = _planes(c_p)

    s_f, cnt = _sc_scatter_feats(dstoff, f2)
    deg = jnp.maximum(cnt[:CN][:NP] + cnt[CN:][:NP], 1.0)
    rdeg = (1.0 / deg)[:, None]

    h = _unplanes(s_f) * rdeg
    f_n = h[:N]

    for _ in range(DEPTH):
        a2 = _planes_tables(h @ Ws.T)
        b2 = _planes_tables(h @ Wd.T)
        s_l = _sc_layer(srcoff, dstoff, c2, a2, b2)
        h = _unplanes(s_l) * rdeg

    return jnp.concatenate([f_n, h[:N]], axis=1)


def _planes_tables(x):
    return jnp.concatenate([x[:, :L], x[:, L:]], axis=0)


# speedup vs baseline: 5.9663x; 1.6285x over previous
import functools

import jax
import jax.numpy as jnp
from jax import lax
from jax.experimental import pallas as pl
from jax.experimental.pallas import tpu as pltpu
from jax.experimental.pallas import tpu_sc as plsc

N = 100000
E = 1600000
IN_FEATS = 16
OUT_FEATS = 32
DEPTH = 3
EPS = 1e-5

L = 16
NTILES = 16
NP = 100096
TRASH = N
K = 256
NB = K // 128
EPT = 100352
NCH = EPT // K
EP = EPT * NTILES
ERB = EP // 128
ART = NP // NTILES
CN = 102400
CNT_ = CN // NTILES
ZROWS = 368
ZREP = ART // ZROWS
CZN = 1600

_f32 = jnp.float32
_i32 = jnp.int32


def _mesh():
    return plsc.VectorSubcoreMesh(core_axis_name="c", subcore_axis_name="s")


def _zero_rows(ref, n):
    def body(i, _):
        ref[i, :] = jnp.zeros((L,), _f32)
        return 0
    lax.fori_loop(0, n, body, 0, unroll=4)


def _sc_scatter_feats(dstoff, f2):
    @functools.partial(
        pl.kernel,
        out_type=(
            jax.ShapeDtypeStruct((2 * NP, L), _f32),
            jax.ShapeDtypeStruct((2 * CN,), _f32),
        ),
        mesh=_mesh(),
        compiler_params=pltpu.CompilerParams(use_tc_tiling_on_sc=False),
        scratch_types=[
            pltpu.VMEM_SHARED((NP, L), _f32),
            pltpu.VMEM_SHARED((CN,), _f32),
            pltpu.VMEM((ZROWS, L), _f32),
            pltpu.VMEM((CZN,), _f32),
            pltpu.VMEM((128,), _f32),
            pltpu.VMEM((NB, 128), _i32),
            pltpu.VMEM((K, L), _f32),
            pltpu.SemaphoreType.DMA,
        ],
    )
    def k(dst_hbm, f_hbm, s_hbm, cnt_hbm, acc, cntacc, zbuf, zbuf1, onesb,
          idxd, buff, sem):
        c = lax.axis_index("c")
        s = lax.axis_index("s")
        _zero_rows(zbuf, ZROWS)
        for j in range(ZREP):
            pltpu.sync_copy(zbuf, acc.at[pl.ds(s * ART + j * ZROWS, ZROWS)])

        def z1(i, _):
            zbuf1[pl.ds(i * L, L)] = jnp.zeros((L,), _f32)
            return 0
        lax.fori_loop(0, CZN // L, z1, 0, unroll=4)
        for i in range(128 // L):
            onesb[pl.ds(i * L, L)] = jnp.ones((L,), _f32)
        for j in range(CNT_ // CZN):
            pltpu.sync_copy(zbuf1, cntacc.at[pl.ds(s * CNT_ + j * CZN, CZN)])
        plsc.subcore_barrier()

        base_row = s * (EPT // 128)

        def chunk(jc, _):
            r0 = base_row + jc * NB
            pltpu.sync_copy(dst_hbm.at[pl.ds(r0, NB)], idxd)
            off = c * EP + s * EPT + jc * K
            cp = pltpu.async_copy(f_hbm.at[pl.ds(off, K)], buff, sem)

            half = NCH // 2

            @pl.when(jnp.logical_and(jc >= c * half, jc < (c + 1) * half))
            def _():
                for k_ in range(NB):
                    pltpu.sync_copy(onesb, cntacc.at[idxd.at[k_]], add=True)

            cp.wait()
            for k_ in range(NB):
                pltpu.sync_copy(buff.at[pl.ds(k_ * 128, 128)],
                                acc.at[idxd.at[k_]], add=True)
            return 0

        lax.fori_loop(0, NCH, chunk, 0)
        plsc.subcore_barrier()
        pltpu.sync_copy(acc.at[pl.ds(s * ART, ART)],
                        s_hbm.at[pl.ds(c * NP + s * ART, ART)])
        pltpu.sync_copy(cntacc.at[pl.ds(s * CNT_, CNT_)],
                        cnt_hbm.at[pl.ds(c * CN + s * CNT_, CNT_)])

    return k(dstoff, f2)


KC = 128
RPT = EPT // KC
HALF = RPT // 2


def _sc_layer(idx2, dstoff, c2, a2, b2):
    @functools.partial(
        pl.kernel,
        out_type=jax.ShapeDtypeStruct((2 * NP, L), _f32),
        mesh=_mesh(),
        compiler_params=pltpu.CompilerParams(use_tc_tiling_on_sc=False),
        scratch_types=[
            pltpu.VMEM_SHARED((NP, L), _f32),
            pltpu.VMEM((ZROWS, L), _f32),
            pltpu.VMEM((2, 128), _i32),
            pltpu.VMEM((2, 128), _i32),
            pltpu.VMEM((1, 128), _i32),
            pltpu.VMEM((1, 128), _i32),
            pltpu.VMEM((KC, L), _f32),
            pltpu.VMEM((KC, L), _f32),
            pltpu.VMEM((KC, L), _f32),
            pltpu.VMEM((KC, L), _f32),
            pltpu.VMEM((KC, L), _f32),
            pltpu.VMEM((KC, L), _f32),
            pltpu.SemaphoreType.DMA,
            pltpu.SemaphoreType.DMA,
            pltpu.SemaphoreType.DMA,
            pltpu.SemaphoreType.DMA,
            pltpu.SemaphoreType.DMA,
            pltpu.SemaphoreType.DMA,
        ],
    )
    def k(idx2_hbm, dsts_hbm, c_hbm, a_hbm, b_hbm, s_hbm, acc, zbuf,
          idxb0, idxb1, sidx0, sidx1, bufa0, bufb0, bufc0, bufa1, bufb1,
          bufc1, semidx0, semidx1, semdat0, semdat1, semsct0, semsct1):
        c = lax.axis_index("c")
        s = lax.axis_index("s")
        _zero_rows(zbuf, ZROWS)
        for j in range(ZREP):
            pltpu.sync_copy(zbuf, acc.at[pl.ds(s * ART + j * ZROWS, ZROWS)])
        plsc.subcore_barrier()

        r0 = s * RPT
        i2b = (c * ERB + r0) * 2
        coff = c * EP + s * EPT

        def issue_data(jc, idxb, sidx, bufa, bufb, bufc, semdat):
            pltpu.async_copy(c_hbm.at[pl.ds(coff + jc * KC, KC)], bufc,
                             semdat)
            pltpu.async_copy(a_hbm.at[idxb.at[0]], bufa, semdat)
            pltpu.async_copy(b_hbm.at[idxb.at[1]], bufb, semdat)
            pltpu.async_copy(dsts_hbm.at[pl.ds(r0 + jc, 1)], sidx, semdat)

        def wait_data(bufa, bufb, bufc, sidx, semdat):
            pltpu.make_async_copy(c_hbm.at[pl.ds(0, KC)], bufa, semdat).wait()
            pltpu.make_async_copy(c_hbm.at[pl.ds(0, KC)], bufb, semdat).wait()
            pltpu.make_async_copy(c_hbm.at[pl.ds(0, KC)], bufc, semdat).wait()
            pltpu.make_async_copy(dsts_hbm.at[pl.ds(0, 1)], sidx,
                                  semdat).wait()

        def issue_idx(jc, idxb, semidx):
            pltpu.async_copy(idx2_hbm.at[pl.ds(i2b + jc * 2, 2)], idxb,
                             semidx)

        def wait_idx(idxb, semidx):
            pltpu.make_async_copy(idx2_hbm.at[pl.ds(0, 2)], idxb,
                                  semidx).wait()

        def wait_sct(bufc, semsct):
            pltpu.make_async_copy(c_hbm.at[pl.ds(0, KC)], bufc, semsct).wait()

        def compute(bufa, bufb, bufc):
            def body(i, _):
                y = bufa[i, :] + bufb[i, :] + bufc[i, :]
                bufc[i, :] = jnp.maximum(y, 0.0)
                return 0
            lax.fori_loop(0, KC, body, 0, unroll=8)

        pltpu.sync_copy(idx2_hbm.at[pl.ds(i2b, 2)], idxb0)
        issue_data(0, idxb0, sidx0, bufa0, bufb0, bufc0, semdat0)
        pltpu.sync_copy(idx2_hbm.at[pl.ds(i2b + 2, 2)], idxb1)
        issue_data(1, idxb1, sidx1, bufa1, bufb1, bufc1, semdat1)

        def step(i, _):
            more = i < HALF - 1
            wait_data(bufa0, bufb0, bufc0, sidx0, semdat0)
            compute(bufa0, bufb0, bufc0)
            pltpu.async_copy(bufc0, acc.at[sidx0.at[0]], semsct0, add=True)

            @pl.when(more)
            def _():
                issue_idx(2 * i + 2, idxb0, semidx0)

            wait_data(bufa1, bufb1, bufc1, sidx1, semdat1)
            compute(bufa1, bufb1, bufc1)
            pltpu.async_copy(bufc1, acc.at[sidx1.at[0]], semsct1, add=True)

            @pl.when(more)
            def _():
                issue_idx(2 * i + 3, idxb1, semidx1)
                wait_idx(idxb0, semidx0)
                wait_sct(bufc0, semsct0)
                issue_data(2 * i + 2, idxb0, sidx0, bufa0, bufb0, bufc0,
                           semdat0)
                wait_idx(idxb1, semidx1)
                wait_sct(bufc1, semsct1)
                issue_data(2 * i + 3, idxb1, sidx1, bufa1, bufb1, bufc1,
                           semdat1)
            return 0

        lax.fori_loop(0, HALF, step, 0)
        wait_sct(bufc0, semsct0)
        wait_sct(bufc1, semsct1)
        plsc.subcore_barrier()
        pltpu.sync_copy(acc.at[pl.ds(s * ART, ART)],
                        s_hbm.at[pl.ds(c * NP + s * ART, ART)])

    return k(idx2, dstoff, c2, a2, b2)


E8 = E // 8
EP8 = EP // 8
NP8 = NP // 8
BES = 2000
BEE = 2048
BNN = 3128


def _bdiag(m, reps=8):
    a, b = m.shape
    eye = jnp.eye(reps, dtype=m.dtype)
    return jnp.einsum("km,ab->kamb", eye, m).reshape(reps * a, reps * b)


def _tc_stats(e8):
    def body(e_ref, o_ref):
        i = pl.program_id(0)
        eb = e_ref[...]

        @pl.when(i == 0)
        def _():
            o_ref[...] = jnp.zeros_like(o_ref)

        o_ref[0, :] += jnp.sum(eb, axis=0)
        o_ref[1, :] += jnp.sum(eb * eb, axis=0)

    return pl.pallas_call(
        body,
        grid=(E8 // BES,),
        in_specs=[pl.BlockSpec((BES, 128), lambda i: (i, 0))],
        out_specs=pl.BlockSpec((2, 128), lambda i: (0, 0)),
        out_shape=jax.ShapeDtypeStruct((2, 128), _f32),
    )(e8)


def _tc_edge(e8p, epp, d1, b1r, s0, s1, d2a, d2b, wpa, wpb, cb0, cb1):
    def body(e_ref, ep_ref, d1_ref, b1_ref, s0_ref, s1_ref, d2a_ref,
             d2b_ref, wpa_ref, wpb_ref, cb0_ref, cb1_ref, f_ref, c_ref):
        eb = e_ref[...]
        f = jnp.maximum(
            jnp.dot(eb, d1_ref[...], preferred_element_type=_f32)
            + b1_ref[...], 0.0)
        ep8 = ep_ref[...]
        c0 = (jnp.dot(f, d2a_ref[...], preferred_element_type=_f32)
              + jnp.dot(ep8, wpa_ref[...], preferred_element_type=_f32)
              + cb0_ref[...])
        c1 = (jnp.dot(f, d2b_ref[...], preferred_element_type=_f32)
              + jnp.dot(ep8, wpb_ref[...], preferred_element_type=_f32)
              + cb1_ref[...])
        f_ref[0, :, :] = jnp.dot(f, s0_ref[...], preferred_element_type=_f32)
        f_ref[1, :, :] = jnp.dot(f, s1_ref[...], preferred_element_type=_f32)
        c_ref[0, :, :] = c0
        c_ref[1, :, :] = c1

    cmap = lambda i: (0, 0)
    f2, c2 = pl.pallas_call(
        body,
        grid=(EP8 // BEE,),
        in_specs=[
            pl.BlockSpec((BEE, 128), lambda i: (i, 0)),
            pl.BlockSpec((BEE, 8), lambda i: (i, 0)),
            pl.BlockSpec((128, 256), cmap),
            pl.BlockSpec((1, 256), cmap),
            pl.BlockSpec((256, 128), cmap),
            pl.BlockSpec((256, 128), cmap),
            pl.BlockSpec((256, 128), cmap),
            pl.BlockSpec((256, 128), cmap),
            pl.BlockSpec((8, 128), cmap),
            pl.BlockSpec((8, 128), cmap),
            pl.BlockSpec((1, 128), cmap),
            pl.BlockSpec((1, 128), cmap),
        ],
        out_specs=[
            pl.BlockSpec((2, BEE, 128), lambda i: (0, i, 0)),
            pl.BlockSpec((2, BEE, 128), lambda i: (0, i, 0)),
        ],
        out_shape=[
            jax.ShapeDtypeStruct((2, EP8, 128), _f32),
            jax.ShapeDtypeStruct((2, EP8, 128), _f32),
        ],
    )(e8p, epp, d1, b1r, s0, s1, d2a, d2b, wpa, wpb, cb0, cb1)
    return f2.reshape(2 * EP, L), c2.reshape(2 * EP, L)


def _tc_node(s2, rdp, ga, gb):
    s3 = s2.reshape(2, NP8, 128)

    def body(s_ref, r_ref, ga_ref, gb_ref, h_ref, a_ref, b_ref):
        r = r_ref[...]
        h0 = s_ref[0, :, :] * r
        h1 = s_ref[1, :, :] * r
        hcat = jnp.concatenate([h0, h1], axis=1)
        a = jnp.dot(hcat, ga_ref[...], preferred_element_type=_f32)
        b = jnp.dot(hcat, gb_ref[...], preferred_element_type=_f32)
        h_ref[0, :, :] = h0
        h_ref[1, :, :] = h1
        a_ref[0, :, :] = a[:, :128]
        a_ref[1, :, :] = a[:, 128:]
        b_ref[0, :, :] = b[:, :128]
        b_ref[1, :, :] = b[:, 128:]

    cmap = lambda i: (0, 0)
    outs = pl.pallas_call(
        body,
        grid=(NP8 // BNN,),
        in_specs=[
            pl.BlockSpec((2, BNN, 128), lambda i: (0, i, 0)),
            pl.BlockSpec((BNN, 128), lambda i: (i, 0)),
            pl.BlockSpec((256, 256), cmap),
            pl.BlockSpec((256, 256), cmap),
        ],
        out_specs=[
            pl.BlockSpec((2, BNN, 128), lambda i: (0, i, 0)),
            pl.BlockSpec((2, BNN, 128), lambda i: (0, i, 0)),
            pl.BlockSpec((2, BNN, 128), lambda i: (0, i, 0)),
        ],
        out_shape=[
            jax.ShapeDtypeStruct((2, NP8, 128), _f32),
            jax.ShapeDtypeStruct((2, NP8, 128), _f32),
            jax.ShapeDtypeStruct((2, NP8, 128), _f32),
        ],
    )(s3, rdp, ga, gb)
    return tuple(o.reshape(2 * NP, L) for o in outs)


def _planes(x):
    return jnp.concatenate([x[:, :L], x[:, L:]], axis=0)


def _unplanes(sp):
    return jnp.concatenate([sp[:NP], sp[NP:]], axis=1)


def kernel(e, edge_index, edge_p, bn_gamma, bn_beta, Wi, bi, Wh, bh):
    src = edge_index[0]
    dst = edge_index[1]

    st = _tc_stats(e.reshape(E8, 128))
    mean = st[0].reshape(8, L).sum(axis=0) / E
    var = st[1].reshape(8, L).sum(axis=0) / E - mean * mean
    scale = bn_gamma / jnp.sqrt(var + EPS)
    shift = bn_beta - mean * scale
    w1t = (Wi * scale[None, :]).T
    b1 = Wi @ shift + bi

    wst = Wh[:, :OUT_FEATS].T
    wft = Wh[:, OUT_FEATS:2 * OUT_FEATS].T
    wp = Wh[:, 2 * OUT_FEATS]
    wdt = Wh[:, 2 * OUT_FEATS + 1:].T

    d1 = _bdiag(w1t)
    b1r = jnp.tile(b1, 8)[None, :]
    sel = jnp.eye(OUT_FEATS, dtype=_f32)
    s0 = _bdiag(sel[:, :L])
    s1 = _bdiag(sel[:, L:])
    d2a = _bdiag(wft[:, :L])
    d2b = _bdiag(wft[:, L:])
    wpa = _bdiag(wp[None, :L])
    wpb = _bdiag(wp[None, L:])
    cb0 = jnp.tile(bh[:L], 8)[None, :]
    cb1 = jnp.tile(bh[L:], 8)[None, :]
    ga = jnp.concatenate(
        [_bdiag(wst[:L, :L]), _bdiag(wst[L:, :L])], axis=0)
    ga = jnp.concatenate([ga, jnp.concatenate(
        [_bdiag(wst[:L, L:]), _bdiag(wst[L:, L:])], axis=0)], axis=1)
    gb = jnp.concatenate(
        [_bdiag(wdt[:L, :L]), _bdiag(wdt[L:, :L])], axis=0)
    gb = jnp.concatenate([gb, jnp.concatenate(
        [_bdiag(wdt[:L, L:]), _bdiag(wdt[L:, L:])], axis=0)], axis=1)

    pad = EP - E
    e8p = jnp.pad(e, ((0, pad), (0, 0))).reshape(EP8, 128)
    epp = jnp.pad(edge_p[:, 0], (0, pad)).reshape(EP8, 8)
    f2, c2 = _tc_edge(e8p, epp, d1, b1r, s0, s1, d2a, d2b, wpa, wpb,
                      cb0, cb1)

    src_p = jnp.pad(src, (0, pad))
    dst_p = jnp.pad(dst, (0, pad), constant_values=TRASH)
    offs = jnp.array([0, NP], _i32)
    srcoff = (src_p[None, :] + offs[:, None]).reshape(2 * ERB, 128)
    dstoff = (dst_p[None, :] + offs[:, None]).reshape(2 * ERB, 128)
    idx2 = jnp.stack([srcoff.reshape(2, ERB, 128),
                      dstoff.reshape(2, ERB, 128)],
                     axis=2).reshape(4 * ERB, 128)

    s_f, cnt = _sc_scatter_feats(dstoff, f2)
    deg = jnp.maximum(cnt[:CN][:NP] + cnt[CN:][:NP], 1.0)
    rdp = jnp.broadcast_to((1.0 / deg)[:, None], (NP, L)).reshape(NP8, 128)

    h2, a2, b2 = _tc_node(s_f, rdp, ga, gb)
    f_n = _unplanes(h2)[:N]

    for _ in range(DEPTH):
        s_l = _sc_layer(idx2, dstoff, c2, a2, b2)
        h2, a2, b2 = _tc_node(s_l, rdp, ga, gb)

    return jnp.concatenate([f_n, _unplanes(h2)[:N]], axis=1)


def _planes_tables(x):
    return jnp.concatenate([x[:, :L], x[:, L:]], axis=0)


# speedup vs baseline: 7.3419x; 1.2306x over previous
import functools

import jax
import jax.numpy as jnp
from jax import lax
from jax.experimental import pallas as pl
from jax.experimental.pallas import tpu as pltpu
from jax.experimental.pallas import tpu_sc as plsc

N = 100000
E = 1600000
IN_FEATS = 16
OUT_FEATS = 32
DEPTH = 3
EPS = 1e-5

L = 16
NTILES = 16
NP = 100096
TRASH = N
K = 256
NB = K // 128
EPT = 100352
NCH = EPT // K
EP = EPT * NTILES
ERB = EP // 128
ART = NP // NTILES
CN = 102400
CNT_ = CN // NTILES
ZROWS = 368
ZREP = ART // ZROWS
CZN = 1600

_f32 = jnp.float32
_i32 = jnp.int32


def _mesh():
    return plsc.VectorSubcoreMesh(core_axis_name="c", subcore_axis_name="s")


def _zero_rows(ref, n):
    def body(i, _):
        ref[i, :] = jnp.zeros((L,), _f32)
        return 0
    lax.fori_loop(0, n, body, 0, unroll=4)


def _sc_scatter_feats(dstoff, f2):
    @functools.partial(
        pl.kernel,
        out_type=(
            jax.ShapeDtypeStruct((2 * NP, L), _f32),
            jax.ShapeDtypeStruct((2 * CN,), _f32),
        ),
        mesh=_mesh(),
        compiler_params=pltpu.CompilerParams(use_tc_tiling_on_sc=False),
        scratch_types=[
            pltpu.VMEM_SHARED((NP, L), _f32),
            pltpu.VMEM_SHARED((CN,), _f32),
            pltpu.VMEM((ZROWS, L), _f32),
            pltpu.VMEM((CZN,), _f32),
            pltpu.VMEM((128,), _f32),
            pltpu.VMEM((1, 128), _i32),
            pltpu.VMEM((1, 128), _i32),
            pltpu.VMEM((KC, L), _f32),
            pltpu.VMEM((KC, L), _f32),
            pltpu.SemaphoreType.DMA,
            pltpu.SemaphoreType.DMA,
            pltpu.SemaphoreType.DMA,
            pltpu.SemaphoreType.DMA,
            pltpu.SemaphoreType.DMA,
            pltpu.SemaphoreType.DMA,
        ],
    )
    def k(dst_hbm, f_hbm, s_hbm, cnt_hbm, acc, cntacc, zbuf, zbuf1, onesb,
          sidx0, sidx1, buff0, buff1, semdat0, semdat1, semsct0, semsct1,
          semcnt0, semcnt1):
        c = lax.axis_index("c")
        s = lax.axis_index("s")
        _zero_rows(zbuf, ZROWS)
        for j in range(ZREP):
            pltpu.sync_copy(zbuf, acc.at[pl.ds(s * ART + j * ZROWS, ZROWS)])

        def z1(i, _):
            zbuf1[pl.ds(i * L, L)] = jnp.zeros((L,), _f32)
            return 0
        lax.fori_loop(0, CZN // L, z1, 0, unroll=4)
        for i in range(128 // L):
            onesb[pl.ds(i * L, L)] = jnp.ones((L,), _f32)
        for j in range(CNT_ // CZN):
            pltpu.sync_copy(zbuf1, cntacc.at[pl.ds(s * CNT_ + j * CZN, CZN)])
        plsc.subcore_barrier()

        r0 = s * RPT
        foff = c * EP + s * EPT
        hlo = c * (RPT // 2)
        hhi = (c + 1) * (RPT // 2)

        def cnt_cond(jc):
            return jnp.logical_and(jc >= hlo, jc < hhi)

        def issue_data(jc, sidx, buff, semdat):
            pltpu.async_copy(f_hbm.at[pl.ds(foff + jc * KC, KC)], buff,
                             semdat)
            pltpu.async_copy(dst_hbm.at[pl.ds(r0 + jc, 1)], sidx, semdat)

        def wait_data(buff, sidx, semdat):
            pltpu.make_async_copy(f_hbm.at[pl.ds(0, KC)], buff,
                                  semdat).wait()
            pltpu.make_async_copy(dst_hbm.at[pl.ds(0, 1)], sidx,
                                  semdat).wait()

        def wait_sct(buff, semsct):
            pltpu.make_async_copy(f_hbm.at[pl.ds(0, KC)], buff,
                                  semsct).wait()

        def wait_cnt(semcnt):
            pltpu.make_async_copy(cnt_hbm.at[pl.ds(0, 128)], onesb,
                                  semcnt).wait()

        def scatter(jc, sidx, buff, semsct, semcnt):
            pltpu.async_copy(buff, acc.at[sidx.at[0]], semsct, add=True)

            @pl.when(cnt_cond(jc))
            def _():
                pltpu.async_copy(onesb, cntacc.at[sidx.at[0]], semcnt,
                                 add=True)

        issue_data(0, sidx0, buff0, semdat0)
        issue_data(1, sidx1, buff1, semdat1)

        def step(i, _):
            more = i < HALF - 1
            wait_data(buff0, sidx0, semdat0)
            scatter(2 * i, sidx0, buff0, semsct0, semcnt0)
            wait_data(buff1, sidx1, semdat1)
            scatter(2 * i + 1, sidx1, buff1, semsct1, semcnt1)

            @pl.when(more)
            def _():
                wait_sct(buff0, semsct0)

                @pl.when(cnt_cond(2 * i))
                def _():
                    wait_cnt(semcnt0)

                issue_data(2 * i + 2, sidx0, buff0, semdat0)
                wait_sct(buff1, semsct1)

                @pl.when(cnt_cond(2 * i + 1))
                def _():
                    wait_cnt(semcnt1)

                issue_data(2 * i + 3, sidx1, buff1, semdat1)
            return 0

        lax.fori_loop(0, HALF, step, 0)
        wait_sct(buff0, semsct0)
        wait_sct(buff1, semsct1)

        @pl.when(cnt_cond(RPT - 2))
        def _():
            wait_cnt(semcnt0)

        @pl.when(cnt_cond(RPT - 1))
        def _():
            wait_cnt(semcnt1)

        plsc.subcore_barrier()
        pltpu.sync_copy(acc.at[pl.ds(s * ART, ART)],
                        s_hbm.at[pl.ds(c * NP + s * ART, ART)])
        pltpu.sync_copy(cntacc.at[pl.ds(s * CNT_, CNT_)],
                        cnt_hbm.at[pl.ds(c * CN + s * CNT_, CNT_)])

    return k(dstoff, f2)


KC = 128
RPT = EPT // KC
HALF = RPT // 2


def _sc_layer(idx2, dstoff, c2, a2, b2):
    @functools.partial(
        pl.kernel,
        out_type=jax.ShapeDtypeStruct((2 * NP, L), _f32),
        mesh=_mesh(),
        compiler_params=pltpu.CompilerParams(use_tc_tiling_on_sc=False),
        scratch_types=[
            pltpu.VMEM_SHARED((NP, L), _f32),
            pltpu.VMEM((ZROWS, L), _f32),
            pltpu.VMEM((2, 128), _i32),
            pltpu.VMEM((2, 128), _i32),
            pltpu.VMEM((1, 128), _i32),
            pltpu.VMEM((1, 128), _i32),
            pltpu.VMEM((KC, L), _f32),
            pltpu.VMEM((KC, L), _f32),
            pltpu.VMEM((KC, L), _f32),
            pltpu.VMEM((KC, L), _f32),
            pltpu.VMEM((KC, L), _f32),
            pltpu.VMEM((KC, L), _f32),
            pltpu.SemaphoreType.DMA,
            pltpu.SemaphoreType.DMA,
            pltpu.SemaphoreType.DMA,
            pltpu.SemaphoreType.DMA,
            pltpu.SemaphoreType.DMA,
            pltpu.SemaphoreType.DMA,
        ],
    )
    def k(idx2_hbm, dsts_hbm, c_hbm, a_hbm, b_hbm, s_hbm, acc, zbuf,
          idxb0, idxb1, sidx0, sidx1, bufa0, bufb0, bufc0, bufa1, bufb1,
          bufc1, semidx0, semidx1, semdat0, semdat1, semsct0, semsct1):
        c = lax.axis_index("c")
        s = lax.axis_index("s")
        _zero_rows(zbuf, ZROWS)
        for j in range(ZREP):
            pltpu.sync_copy(zbuf, acc.at[pl.ds(s * ART + j * ZROWS, ZROWS)])
        plsc.subcore_barrier()

        r0 = s * RPT
        i2b = (c * ERB + r0) * 2
        coff = c * EP + s * EPT

        def issue_data(jc, idxb, sidx, bufa, bufb, bufc, semdat):
            pltpu.async_copy(c_hbm.at[pl.ds(coff + jc * KC, KC)], bufc,
                             semdat)
            pltpu.async_copy(a_hbm.at[idxb.at[0]], bufa, semdat)
            pltpu.async_copy(b_hbm.at[idxb.at[1]], bufb, semdat)
            pltpu.async_copy(dsts_hbm.at[pl.ds(r0 + jc, 1)], sidx, semdat)

        def wait_data(bufa, bufb, bufc, sidx, semdat):
            pltpu.make_async_copy(c_hbm.at[pl.ds(0, KC)], bufa, semdat).wait()
            pltpu.make_async_copy(c_hbm.at[pl.ds(0, KC)], bufb, semdat).wait()
            pltpu.make_async_copy(c_hbm.at[pl.ds(0, KC)], bufc, semdat).wait()
            pltpu.make_async_copy(dsts_hbm.at[pl.ds(0, 1)], sidx,
                                  semdat).wait()

        def issue_idx(jc, idxb, semidx):
            pltpu.async_copy(idx2_hbm.at[pl.ds(i2b + jc * 2, 2)], idxb,
                             semidx)

        def wait_idx(idxb, semidx):
            pltpu.make_async_copy(idx2_hbm.at[pl.ds(0, 2)], idxb,
                                  semidx).wait()

        def wait_sct(bufc, semsct):
            pltpu.make_async_copy(c_hbm.at[pl.ds(0, KC)], bufc, semsct).wait()

        def compute(bufa, bufb, bufc):
            def body(i, _):
                y = bufa[i, :] + bufb[i, :] + bufc[i, :]
                bufc[i, :] = jnp.maximum(y, 0.0)
                return 0
            lax.fori_loop(0, KC, body, 0, unroll=8)

        pltpu.sync_copy(idx2_hbm.at[pl.ds(i2b, 2)], idxb0)
        issue_data(0, idxb0, sidx0, bufa0, bufb0, bufc0, semdat0)
        pltpu.sync_copy(idx2_hbm.at[pl.ds(i2b + 2, 2)], idxb1)
        issue_data(1, idxb1, sidx1, bufa1, bufb1, bufc1, semdat1)

        def step(i, _):
            more = i < HALF - 1
            wait_data(bufa0, bufb0, bufc0, sidx0, semdat0)
            compute(bufa0, bufb0, bufc0)
            pltpu.async_copy(bufc0, acc.at[sidx0.at[0]], semsct0, add=True)

            @pl.when(more)
            def _():
                issue_idx(2 * i + 2, idxb0, semidx0)

            wait_data(bufa1, bufb1, bufc1, sidx1, semdat1)
            compute(bufa1, bufb1, bufc1)
            pltpu.async_copy(bufc1, acc.at[sidx1.at[0]], semsct1, add=True)

            @pl.when(more)
            def _():
                issue_idx(2 * i + 3, idxb1, semidx1)
                wait_idx(idxb0, semidx0)
                wait_sct(bufc0, semsct0)
                issue_data(2 * i + 2, idxb0, sidx0, bufa0, bufb0, bufc0,
                           semdat0)
                wait_idx(idxb1, semidx1)
                wait_sct(bufc1, semsct1)
                issue_data(2 * i + 3, idxb1, sidx1, bufa1, bufb1, bufc1,
                           semdat1)
            return 0

        lax.fori_loop(0, HALF, step, 0)
        wait_sct(bufc0, semsct0)
        wait_sct(bufc1, semsct1)
        plsc.subcore_barrier()
        pltpu.sync_copy(acc.at[pl.ds(s * ART, ART)],
                        s_hbm.at[pl.ds(c * NP + s * ART, ART)])

    return k(idx2, dstoff, c2, a2, b2)


E8 = E // 8
EP8 = EP // 8
NP8 = NP // 8
BES = 2000
BEE = 2048
BNN = 3128


def _bdiag(m, reps=8):
    a, b = m.shape
    eye = jnp.eye(reps, dtype=m.dtype)
    return jnp.einsum("km,ab->kamb", eye, m).reshape(reps * a, reps * b)


def _tc_stats(e8p):
    def body(e_ref, o_ref):
        i = pl.program_id(0)
        eb = e_ref[...]

        @pl.when(i == 0)
        def _():
            o_ref[...] = jnp.zeros_like(o_ref)

        o_ref[0, :] += jnp.sum(eb, axis=0)
        o_ref[1, :] += jnp.sum(eb * eb, axis=0)

    return pl.pallas_call(
        body,
        grid=(EP8 // BEE,),
        in_specs=[pl.BlockSpec((BEE, 128), lambda i: (i, 0))],
        out_specs=pl.BlockSpec((2, 128), lambda i: (0, 0)),
        out_shape=jax.ShapeDtypeStruct((2, 128), _f32),
    )(e8p)


def _tc_edge(e8p, epp, d1, b1r, s0, s1, d2a, d2b, wpa, wpb, cb0, cb1):
    def body(e_ref, ep_ref, d1_ref, b1_ref, s0_ref, s1_ref, d2a_ref,
             d2b_ref, wpa_ref, wpb_ref, cb0_ref, cb1_ref, f_ref, c_ref):
        eb = e_ref[...]
        f = jnp.maximum(
            jnp.dot(eb, d1_ref[...], preferred_element_type=_f32)
            + b1_ref[...], 0.0)
        ep8 = ep_ref[...]
        c0 = (jnp.dot(f, d2a_ref[...], preferred_element_type=_f32)
              + jnp.dot(ep8, wpa_ref[...], preferred_element_type=_f32)
              + cb0_ref[...])
        c1 = (jnp.dot(f, d2b_ref[...], preferred_element_type=_f32)
              + jnp.dot(ep8, wpb_ref[...], preferred_element_type=_f32)
              + cb1_ref[...])
        f_ref[0, :, :] = jnp.dot(f, s0_ref[...], preferred_element_type=_f32)
        f_ref[1, :, :] = jnp.dot(f, s1_ref[...], preferred_element_type=_f32)
        c_ref[0, :, :] = c0
        c_ref[1, :, :] = c1

    cmap = lambda i: (0, 0)
    f2, c2 = pl.pallas_call(
        body,
        grid=(EP8 // BEE,),
        in_specs=[
            pl.BlockSpec((BEE, 128), lambda i: (i, 0)),
            pl.BlockSpec((BEE, 8), lambda i: (i, 0)),
            pl.BlockSpec((128, 256), cmap),
            pl.BlockSpec((1, 256), cmap),
            pl.BlockSpec((256, 128), cmap),
            pl.BlockSpec((256, 128), cmap),
            pl.BlockSpec((256, 128), cmap),
            pl.BlockSpec((256, 128), cmap),
            pl.BlockSpec((8, 128), cmap),
            pl.BlockSpec((8, 128), cmap),
            pl.BlockSpec((1, 128), cmap),
            pl.BlockSpec((1, 128), cmap),
        ],
        out_specs=[
            pl.BlockSpec((2, BEE, 128), lambda i: (0, i, 0)),
            pl.BlockSpec((2, BEE, 128), lambda i: (0, i, 0)),
        ],
        out_shape=[
            jax.ShapeDtypeStruct((2, EP8, 128), _f32),
            jax.ShapeDtypeStruct((2, EP8, 128), _f32),
        ],
    )(e8p, epp, d1, b1r, s0, s1, d2a, d2b, wpa, wpb, cb0, cb1)
    return f2.reshape(2 * EP, L), c2.reshape(2 * EP, L)


def _tc_node(s2, rdp, ga, gb):
    s3 = s2.reshape(2, NP8, 128)

    def body(s_ref, r_ref, ga_ref, gb_ref, h_ref, a_ref, b_ref):
        r = r_ref[...]
        h0 = s_ref[0, :, :] * r
        h1 = s_ref[1, :, :] * r
        hcat = jnp.concatenate([h0, h1], axis=1)
        a = jnp.dot(hcat, ga_ref[...], preferred_element_type=_f32)
        b = jnp.dot(hcat, gb_ref[...], preferred_element_type=_f32)
        h_ref[0, :, :] = h0
        h_ref[1, :, :] = h1
        a_ref[0, :, :] = a[:, :128]
        a_ref[1, :, :] = a[:, 128:]
        b_ref[0, :, :] = b[:, :128]
        b_ref[1, :, :] = b[:, 128:]

    cmap = lambda i: (0, 0)
    outs = pl.pallas_call(
        body,
        grid=(NP8 // BNN,),
        in_specs=[
            pl.BlockSpec((2, BNN, 128), lambda i: (0, i, 0)),
            pl.BlockSpec((BNN, 128), lambda i: (i, 0)),
            pl.BlockSpec((256, 256), cmap),
            pl.BlockSpec((256, 256), cmap),
        ],
        out_specs=[
            pl.BlockSpec((2, BNN, 128), lambda i: (0, i, 0)),
            pl.BlockSpec((2, BNN, 128), lambda i: (0, i, 0)),
            pl.BlockSpec((2, BNN, 128), lambda i: (0, i, 0)),
        ],
        out_shape=[
            jax.ShapeDtypeStruct((2, NP8, 128), _f32),
            jax.ShapeDtypeStruct((2, NP8, 128), _f32),
            jax.ShapeDtypeStruct((2, NP8, 128), _f32),
        ],
    )(s3, rdp, ga, gb)
    return tuple(o.reshape(2 * NP, L) for o in outs)


def _planes(x):
    return jnp.concatenate([x[:, :L], x[:, L:]], axis=0)


def _unplanes(sp):
    return jnp.concatenate([sp[:NP], sp[NP:]], axis=1)


def kernel(e, edge_index, edge_p, bn_gamma, bn_beta, Wi, bi, Wh, bh):
    src = edge_index[0]
    dst = edge_index[1]

    e8p = jnp.pad(e.reshape(E8, 128), ((0, EP8 - E8), (0, 0)))

    st = _tc_stats(e8p)
    mean = st[0].reshape(8, L).sum(axis=0) / E
    var = st[1].reshape(8, L).sum(axis=0) / E - mean * mean
    scale = bn_gamma / jnp.sqrt(var + EPS)
    shift = bn_beta - mean * scale
    w1t = (Wi * scale[None, :]).T
    b1 = Wi @ shift + bi

    wst = Wh[:, :OUT_FEATS].T
    wft = Wh[:, OUT_FEATS:2 * OUT_FEATS].T
    wp = Wh[:, 2 * OUT_FEATS]
    wdt = Wh[:, 2 * OUT_FEATS + 1:].T

    d1 = _bdiag(w1t)
    b1r = jnp.tile(b1, 8)[None, :]
    sel = jnp.eye(OUT_FEATS, dtype=_f32)
    s0 = _bdiag(sel[:, :L])
    s1 = _bdiag(sel[:, L:])
    d2a = _bdiag(wft[:, :L])
    d2b = _bdiag(wft[:, L:])
    wpa = _bdiag(wp[None, :L])
    wpb = _bdiag(wp[None, L:])
    cb0 = jnp.tile(bh[:L], 8)[None, :]
    cb1 = jnp.tile(bh[L:], 8)[None, :]
    ga = jnp.concatenate(
        [_bdiag(wst[:L, :L]), _bdiag(wst[L:, :L])], axis=0)
    ga = jnp.concatenate([ga, jnp.concatenate(
        [_bdiag(wst[:L, L:]), _bdiag(wst[L:, L:])], axis=0)], axis=1)
    gb = jnp.concatenate(
        [_bdiag(wdt[:L, :L]), _bdiag(wdt[L:, :L])], axis=0)
    gb = jnp.concatenate([gb, jnp.concatenate(
        [_bdiag(wdt[:L, L:]), _bdiag(wdt[L:, L:])], axis=0)], axis=1)

    pad = EP - E
    epp = jnp.pad(edge_p[:, 0], (0, pad)).reshape(EP8, 8)
    f2, c2 = _tc_edge(e8p, epp, d1, b1r, s0, s1, d2a, d2b, wpa, wpb,
                      cb0, cb1)

    src_p = jnp.pad(src, (0, pad))
    dst_p = jnp.pad(dst, (0, pad), constant_values=TRASH)
    offs = jnp.array([0, NP], _i32)
    srcoff = (src_p[None, :] + offs[:, None]).reshape(2 * ERB, 128)
    dstoff = (dst_p[None, :] + offs[:, None]).reshape(2 * ERB, 128)
    idx2 = jnp.stack([srcoff.reshape(2, ERB, 128),
                      dstoff.reshape(2, ERB, 128)],
                     axis=2).reshape(4 * ERB, 128)

    s_f, cnt = _sc_scatter_feats(dstoff, f2)
    deg = jnp.maximum(cnt[:CN][:NP] + cnt[CN:][:NP], 1.0)
    rdp = jnp.broadcast_to((1.0 / deg)[:, None], (NP, L)).reshape(NP8, 128)

    h2, a2, b2 = _tc_node(s_f, rdp, ga, gb)
    f_n = _unplanes(h2)[:N]

    for _ in range(DEPTH):
        s_l = _sc_layer(idx2, dstoff, c2, a2, b2)
        h2, a2, b2 = _tc_node(s_l, rdp, ga, gb)

    return jnp.concatenate([f_n, _unplanes(h2)[:N]], axis=1)


def _planes_tables(x):
    return jnp.concatenate([x[:, :L], x[:, L:]], axis=0)


# speedup vs baseline: 8.8729x; 1.2085x over previous
import functools

import jax
import jax.numpy as jnp
from jax import lax
from jax.experimental import pallas as pl
from jax.experimental.pallas import tpu as pltpu
from jax.experimental.pallas import tpu_sc as plsc

N = 100000
E = 1600000
IN_FEATS = 16
OUT_FEATS = 32
DEPTH = 3
EPS = 1e-5

L = 16
NTILES = 16
NP = 100096
TRASH = N
K = 256
NB = K // 128
EPT = 100352
NCH = EPT // K
EP = EPT * NTILES
ERB = EP // 128
ART = NP // NTILES
CN = 102400
CNT_ = CN // NTILES
ZROWS = 368
ZREP = ART // ZROWS
CZN = 1600

_f32 = jnp.float32
_i32 = jnp.int32


def _mesh():
    return plsc.VectorSubcoreMesh(core_axis_name="c", subcore_axis_name="s")


def _zero_rows(ref, n):
    def body(i, _):
        ref[i, :] = jnp.zeros((L,), _f32)
        return 0
    lax.fori_loop(0, n, body, 0, unroll=4)


def _sc_scatter_feats(dstoff, f2):
    @functools.partial(
        pl.kernel,
        out_type=(
            jax.ShapeDtypeStruct((2 * NP, L), _f32),
            jax.ShapeDtypeStruct((2 * CN,), _f32),
        ),
        mesh=_mesh(),
        compiler_params=pltpu.CompilerParams(use_tc_tiling_on_sc=False),
        scratch_types=[
            pltpu.VMEM_SHARED((NP, L), _f32),
            pltpu.VMEM_SHARED((CN,), _f32),
            pltpu.VMEM((ZROWS, L), _f32),
            pltpu.VMEM((CZN,), _f32),
            pltpu.VMEM((128,), _f32),
            pltpu.VMEM((1, 128), _i32),
            pltpu.VMEM((1, 128), _i32),
            pltpu.VMEM((KC, L), _f32),
            pltpu.VMEM((KC, L), _f32),
            pltpu.SemaphoreType.DMA,
            pltpu.SemaphoreType.DMA,
            pltpu.SemaphoreType.DMA,
            pltpu.SemaphoreType.DMA,
            pltpu.SemaphoreType.DMA,
            pltpu.SemaphoreType.DMA,
        ],
    )
    def k(dst_hbm, f_hbm, s_hbm, cnt_hbm, acc, cntacc, zbuf, zbuf1, onesb,
          sidx0, sidx1, buff0, buff1, semdat0, semdat1, semsct0, semsct1,
          semcnt0, semcnt1):
        c = lax.axis_index("c")
        s = lax.axis_index("s")
        _zero_rows(zbuf, ZROWS)
        for j in range(ZREP):
            pltpu.sync_copy(zbuf, acc.at[pl.ds(s * ART + j * ZROWS, ZROWS)])

        def z1(i, _):
            zbuf1[pl.ds(i * L, L)] = jnp.zeros((L,), _f32)
            return 0
        lax.fori_loop(0, CZN // L, z1, 0, unroll=4)
        for i in range(128 // L):
            onesb[pl.ds(i * L, L)] = jnp.ones((L,), _f32)
        for j in range(CNT_ // CZN):
            pltpu.sync_copy(zbuf1, cntacc.at[pl.ds(s * CNT_ + j * CZN, CZN)])
        plsc.subcore_barrier()

        r0 = s * RPT
        foff = c * EP + s * EPT
        hlo = c * (RPT // 2)
        hhi = (c + 1) * (RPT // 2)

        def cnt_cond(jc):
            return jnp.logical_and(jc >= hlo, jc < hhi)

        def issue_data(jc, sidx, buff, semdat):
            pltpu.async_copy(f_hbm.at[pl.ds(foff + jc * KC, KC)], buff,
                             semdat)
            pltpu.async_copy(dst_hbm.at[pl.ds(r0 + jc, 1)], sidx, semdat)

        def wait_data(buff, sidx, semdat):
            pltpu.make_async_copy(f_hbm.at[pl.ds(0, KC)], buff,
                                  semdat).wait()
            pltpu.make_async_copy(dst_hbm.at[pl.ds(0, 1)], sidx,
                                  semdat).wait()

        def wait_sct(buff, semsct):
            pltpu.make_async_copy(f_hbm.at[pl.ds(0, KC)], buff,
                                  semsct).wait()

        def wait_cnt(semcnt):
            pltpu.make_async_copy(cnt_hbm.at[pl.ds(0, 128)], onesb,
                                  semcnt).wait()

        def scatter(jc, sidx, buff, semsct, semcnt):
            pltpu.async_copy(buff, acc.at[sidx.at[0]], semsct, add=True)

            @pl.when(cnt_cond(jc))
            def _():
                pltpu.async_copy(onesb, cntacc.at[sidx.at[0]], semcnt,
                                 add=True)

        issue_data(0, sidx0, buff0, semdat0)
        issue_data(1, sidx1, buff1, semdat1)

        def step(i, _):
            more = i < HALF - 1
            wait_data(buff0, sidx0, semdat0)
            scatter(2 * i, sidx0, buff0, semsct0, semcnt0)
            wait_data(buff1, sidx1, semdat1)
            scatter(2 * i + 1, sidx1, buff1, semsct1, semcnt1)

            @pl.when(more)
            def _():
                wait_sct(buff0, semsct0)

                @pl.when(cnt_cond(2 * i))
                def _():
                    wait_cnt(semcnt0)

                issue_data(2 * i + 2, sidx0, buff0, semdat0)
                wait_sct(buff1, semsct1)

                @pl.when(cnt_cond(2 * i + 1))
                def _():
                    wait_cnt(semcnt1)

                issue_data(2 * i + 3, sidx1, buff1, semdat1)
            return 0

        lax.fori_loop(0, HALF, step, 0)
        wait_sct(buff0, semsct0)
        wait_sct(buff1, semsct1)

        @pl.when(cnt_cond(RPT - 2))
        def _():
            wait_cnt(semcnt0)

        @pl.when(cnt_cond(RPT - 1))
        def _():
            wait_cnt(semcnt1)

        plsc.subcore_barrier()
        pltpu.sync_copy(acc.at[pl.ds(s * ART, ART)],
                        s_hbm.at[pl.ds(c * NP + s * ART, ART)])
        pltpu.sync_copy(cntacc.at[pl.ds(s * CNT_, CNT_)],
                        cnt_hbm.at[pl.ds(c * CN + s * CNT_, CNT_)])

    return k(dstoff, f2)


KC = 128
RPT = EPT // KC
HALF = RPT // 2


def _sc_layer(idx2, dstoff, c2, a2, b2):
    @functools.partial(
        pl.kernel,
        out_type=jax.ShapeDtypeStruct((2 * NP, L), _f32),
        mesh=_mesh(),
        compiler_params=pltpu.CompilerParams(use_tc_tiling_on_sc=False),
        scratch_types=[
            pltpu.VMEM_SHARED((NP, L), _f32),
            pltpu.VMEM((ZROWS, L), _f32),
            pltpu.VMEM((2, 128), _i32),
            pltpu.VMEM((2, 128), _i32),
            pltpu.VMEM((1, 128), _i32),
            pltpu.VMEM((1, 128), _i32),
            pltpu.VMEM((KC, L), _f32),
            pltpu.VMEM((KC, L), _f32),
            pltpu.VMEM((KC, L), _f32),
            pltpu.VMEM((KC, L), _f32),
            pltpu.VMEM((KC, L), _f32),
            pltpu.VMEM((KC, L), _f32),
            pltpu.SemaphoreType.DMA,
            pltpu.SemaphoreType.DMA,
            pltpu.SemaphoreType.DMA,
            pltpu.SemaphoreType.DMA,
            pltpu.SemaphoreType.DMA,
            pltpu.SemaphoreType.DMA,
        ],
    )
    def k(idx2_hbm, dsts_hbm, c_hbm, a_hbm, b_hbm, s_hbm, acc, zbuf,
          idxb0, idxb1, sidx0, sidx1, bufa0, bufb0, bufc0, bufa1, bufb1,
          bufc1, semidx0, semidx1, semdat0, semdat1, semsct0, semsct1):
        c = lax.axis_index("c")
        s = lax.axis_index("s")
        _zero_rows(zbuf, ZROWS)
        for j in range(ZREP):
            pltpu.sync_copy(zbuf, acc.at[pl.ds(s * ART + j * ZROWS, ZROWS)])
        plsc.subcore_barrier()

        r0 = s * RPT
        i2b = (c * ERB + r0) * 2
        coff = c * EP + s * EPT

        def issue_data(jc, idxb, sidx, bufa, bufb, bufc, semdat):
            pltpu.async_copy(c_hbm.at[pl.ds(coff + jc * KC, KC)], bufc,
                             semdat)
            pltpu.async_copy(a_hbm.at[idxb.at[0]], bufa, semdat)
            pltpu.async_copy(b_hbm.at[idxb.at[1]], bufb, semdat)
            pltpu.async_copy(dsts_hbm.at[pl.ds(r0 + jc, 1)], sidx, semdat)

        def wait_data(bufa, bufb, bufc, sidx, semdat):
            pltpu.make_async_copy(c_hbm.at[pl.ds(0, KC)], bufa, semdat).wait()
            pltpu.make_async_copy(c_hbm.at[pl.ds(0, KC)], bufb, semdat).wait()
            pltpu.make_async_copy(c_hbm.at[pl.ds(0, KC)], bufc, semdat).wait()
            pltpu.make_async_copy(dsts_hbm.at[pl.ds(0, 1)], sidx,
                                  semdat).wait()

        def issue_idx(jc, idxb, semidx):
            pltpu.async_copy(idx2_hbm.at[pl.ds(i2b + jc * 2, 2)], idxb,
                             semidx)

        def wait_idx(idxb, semidx):
            pltpu.make_async_copy(idx2_hbm.at[pl.ds(0, 2)], idxb,
                                  semidx).wait()

        def wait_sct(bufc, semsct):
            pltpu.make_async_copy(c_hbm.at[pl.ds(0, KC)], bufc, semsct).wait()

        def compute(bufa, bufb, bufc):
            bufc[...] = jnp.maximum(bufa[...] + bufb[...] + bufc[...], 0.0)

        pltpu.sync_copy(idx2_hbm.at[pl.ds(i2b, 2)], idxb0)
        issue_data(0, idxb0, sidx0, bufa0, bufb0, bufc0, semdat0)
        pltpu.sync_copy(idx2_hbm.at[pl.ds(i2b + 2, 2)], idxb1)
        issue_data(1, idxb1, sidx1, bufa1, bufb1, bufc1, semdat1)

        def step(i, _):
            more = i < HALF - 1
            wait_data(bufa0, bufb0, bufc0, sidx0, semdat0)
            compute(bufa0, bufb0, bufc0)
            pltpu.async_copy(bufc0, acc.at[sidx0.at[0]], semsct0, add=True)

            @pl.when(more)
            def _():
                issue_idx(2 * i + 2, idxb0, semidx0)

            wait_data(bufa1, bufb1, bufc1, sidx1, semdat1)
            compute(bufa1, bufb1, bufc1)
            pltpu.async_copy(bufc1, acc.at[sidx1.at[0]], semsct1, add=True)

            @pl.when(more)
            def _():
                issue_idx(2 * i + 3, idxb1, semidx1)
                wait_idx(idxb0, semidx0)
                wait_sct(bufc0, semsct0)
                issue_data(2 * i + 2, idxb0, sidx0, bufa0, bufb0, bufc0,
                           semdat0)
                wait_idx(idxb1, semidx1)
                wait_sct(bufc1, semsct1)
                issue_data(2 * i + 3, idxb1, sidx1, bufa1, bufb1, bufc1,
                           semdat1)
            return 0

        lax.fori_loop(0, HALF, step, 0)
        wait_sct(bufc0, semsct0)
        wait_sct(bufc1, semsct1)
        plsc.subcore_barrier()
        pltpu.sync_copy(acc.at[pl.ds(s * ART, ART)],
                        s_hbm.at[pl.ds(c * NP + s * ART, ART)])

    return k(idx2, dstoff, c2, a2, b2)


E8 = E // 8
EP8 = EP // 8
NP8 = NP // 8
BES = 2000
BEE = 2048
BNN = 3128


def _bdiag(m, reps=8):
    a, b = m.shape
    eye = jnp.eye(reps, dtype=m.dtype)
    return jnp.einsum("km,ab->kamb", eye, m).reshape(reps * a, reps * b)


def _tc_stats(e8p):
    def body(e_ref, o_ref):
        i = pl.program_id(0)
        eb = e_ref[...]

        @pl.when(i == 0)
        def _():
            o_ref[...] = jnp.zeros_like(o_ref)

        o_ref[0, :] += jnp.sum(eb, axis=0)
        o_ref[1, :] += jnp.sum(eb * eb, axis=0)

    return pl.pallas_call(
        body,
        grid=(EP8 // BEE,),
        in_specs=[pl.BlockSpec((BEE, 128), lambda i: (i, 0))],
        out_specs=pl.BlockSpec((2, 128), lambda i: (0, 0)),
        out_shape=jax.ShapeDtypeStruct((2, 128), _f32),
    )(e8p)


def _tc_edge(e8p, epp, d1, b1r, s0, s1, d2a, d2b, wpa, wpb, cb0, cb1):
    def body(e_ref, ep_ref, d1_ref, b1_ref, s0_ref, s1_ref, d2a_ref,
             d2b_ref, wpa_ref, wpb_ref, cb0_ref, cb1_ref, f_ref, c_ref):
        eb = e_ref[...]
        f = jnp.maximum(
            jnp.dot(eb, d1_ref[...], preferred_element_type=_f32)
            + b1_ref[...], 0.0)
        ep8 = ep_ref[...]
        c0 = (jnp.dot(f, d2a_ref[...], preferred_element_type=_f32)
              + jnp.dot(ep8, wpa_ref[...], preferred_element_type=_f32)
              + cb0_ref[...])
        c1 = (jnp.dot(f, d2b_ref[...], preferred_element_type=_f32)
              + jnp.dot(ep8, wpb_ref[...], preferred_element_type=_f32)
              + cb1_ref[...])
        f_ref[0, :, :] = jnp.dot(f, s0_ref[...], preferred_element_type=_f32)
        f_ref[1, :, :] = jnp.dot(f, s1_ref[...], preferred_element_type=_f32)
        c_ref[0, :, :] = c0
        c_ref[1, :, :] = c1

    cmap = lambda i: (0, 0)
    f2, c2 = pl.pallas_call(
        body,
        grid=(EP8 // BEE,),
        in_specs=[
            pl.BlockSpec((BEE, 128), lambda i: (i, 0)),
            pl.BlockSpec((BEE, 8), lambda i: (i, 0)),
            pl.BlockSpec((128, 256), cmap),
            pl.BlockSpec((1, 256), cmap),
            pl.BlockSpec((256, 128), cmap),
            pl.BlockSpec((256, 128), cmap),
            pl.BlockSpec((256, 128), cmap),
            pl.BlockSpec((256, 128), cmap),
            pl.BlockSpec((8, 128), cmap),
            pl.BlockSpec((8, 128), cmap),
            pl.BlockSpec((1, 128), cmap),
            pl.BlockSpec((1, 128), cmap),
        ],
        out_specs=[
            pl.BlockSpec((2, BEE, 128), lambda i: (0, i, 0)),
            pl.BlockSpec((2, BEE, 128), lambda i: (0, i, 0)),
        ],
        out_shape=[
            jax.ShapeDtypeStruct((2, EP8, 128), _f32),
            jax.ShapeDtypeStruct((2, EP8, 128), _f32),
        ],
    )(e8p, epp, d1, b1r, s0, s1, d2a, d2b, wpa, wpb, cb0, cb1)
    return f2.reshape(2 * EP, L), c2.reshape(2 * EP, L)


def _tc_node(s2, rdp, ga, gb):
    s3 = s2.reshape(2, NP8, 128)

    def body(s_ref, r_ref, ga_ref, gb_ref, h_ref, a_ref, b_ref):
        r = r_ref[...]
        h0 = s_ref[0, :, :] * r
        h1 = s_ref[1, :, :] * r
        hcat = jnp.concatenate([h0, h1], axis=1)
        a = jnp.dot(hcat, ga_ref[...], preferred_element_type=_f32)
        b = jnp.dot(hcat, gb_ref[...], preferred_element_type=_f32)
        h_ref[0, :, :] = h0
        h_ref[1, :, :] = h1
        a_ref[0, :, :] = a[:, :128]
        a_ref[1, :, :] = a[:, 128:]
        b_ref[0, :, :] = b[:, :128]
        b_ref[1, :, :] = b[:, 128:]

    cmap = lambda i: (0, 0)
    outs = pl.pallas_call(
        body,
        grid=(NP8 // BNN,),
        in_specs=[
            pl.BlockSpec((2, BNN, 128), lambda i: (0, i, 0)),
            pl.BlockSpec((BNN, 128), lambda i: (i, 0)),
            pl.BlockSpec((256, 256), cmap),
            pl.BlockSpec((256, 256), cmap),
        ],
        out_specs=[
            pl.BlockSpec((2, BNN, 128), lambda i: (0, i, 0)),
            pl.BlockSpec((2, BNN, 128), lambda i: (0, i, 0)),
            pl.BlockSpec((2, BNN, 128), lambda i: (0, i, 0)),
        ],
        out_shape=[
            jax.ShapeDtypeStruct((2, NP8, 128), _f32),
            jax.ShapeDtypeStruct((2, NP8, 128), _f32),
            jax.ShapeDtypeStruct((2, NP8, 128), _f32),
        ],
    )(s3, rdp, ga, gb)
    return tuple(o.reshape(2 * NP, L) for o in outs)


def _planes(x):
    return jnp.concatenate([x[:, :L], x[:, L:]], axis=0)


def _unplanes(sp):
    return jnp.concatenate([sp[:NP], sp[NP:]], axis=1)


def kernel(e, edge_index, edge_p, bn_gamma, bn_beta, Wi, bi, Wh, bh):
    src = edge_index[0]
    dst = edge_index[1]

    e8p = jnp.pad(e.reshape(E8, 128), ((0, EP8 - E8), (0, 0)))

    st = _tc_stats(e8p)
    mean = st[0].reshape(8, L).sum(axis=0) / E
    var = st[1].reshape(8, L).sum(axis=0) / E - mean * mean
    scale = bn_gamma / jnp.sqrt(var + EPS)
    shift = bn_beta - mean * scale
    w1t = (Wi * scale[None, :]).T
    b1 = Wi @ shift + bi

    wst = Wh[:, :OUT_FEATS].T
    wft = Wh[:, OUT_FEATS:2 * OUT_FEATS].T
    wp = Wh[:, 2 * OUT_FEATS]
    wdt = Wh[:, 2 * OUT_FEATS + 1:].T

    d1 = _bdiag(w1t)
    b1r = jnp.tile(b1, 8)[None, :]
    sel = jnp.eye(OUT_FEATS, dtype=_f32)
    s0 = _bdiag(sel[:, :L])
    s1 = _bdiag(sel[:, L:])
    d2a = _bdiag(wft[:, :L])
    d2b = _bdiag(wft[:, L:])
    wpa = _bdiag(wp[None, :L])
    wpb = _bdiag(wp[None, L:])
    cb0 = jnp.tile(bh[:L], 8)[None, :]
    cb1 = jnp.tile(bh[L:], 8)[None, :]
    ga = jnp.concatenate(
        [_bdiag(wst[:L, :L]), _bdiag(wst[L:, :L])], axis=0)
    ga = jnp.concatenate([ga, jnp.concatenate(
        [_bdiag(wst[:L, L:]), _bdiag(wst[L:, L:])], axis=0)], axis=1)
    gb = jnp.concatenate(
        [_bdiag(wdt[:L, :L]), _bdiag(wdt[L:, :L])], axis=0)
    gb = jnp.concatenate([gb, jnp.concatenate(
        [_bdiag(wdt[:L, L:]), _bdiag(wdt[L:, L:])], axis=0)], axis=1)

    pad = EP - E
    epp = jnp.pad(edge_p[:, 0], (0, pad)).reshape(EP8, 8)
    f2, c2 = _tc_edge(e8p, epp, d1, b1r, s0, s1, d2a, d2b, wpa, wpb,
                      cb0, cb1)

    src_p = jnp.pad(src, (0, pad))
    dst_p = jnp.pad(dst, (0, pad), constant_values=TRASH)
    offs = jnp.array([0, NP], _i32)
    srcoff = (src_p[None, :] + offs[:, None]).reshape(2 * ERB, 128)
    dstoff = (dst_p[None, :] + offs[:, None]).reshape(2 * ERB, 128)
    idx2 = jnp.stack([srcoff.reshape(2, ERB, 128),
                      dstoff.reshape(2, ERB, 128)],
                     axis=2).reshape(4 * ERB, 128)

    s_f, cnt = _sc_scatter_feats(dstoff, f2)
    deg = jnp.maximum(cnt[:CN][:NP] + cnt[CN:][:NP], 1.0)
    rdp = jnp.broadcast_to((1.0 / deg)[:, None], (NP, L)).reshape(NP8, 128)

    h2, a2, b2 = _tc_node(s_f, rdp, ga, gb)
    f_n = _unplanes(h2)[:N]

    for _ in range(DEPTH):
        s_l = _sc_layer(idx2, dstoff, c2, a2, b2)
        h2, a2, b2 = _tc_node(s_l, rdp, ga, gb)

    return jnp.concatenate([f_n, _unplanes(h2)[:N]], axis=1)


def _planes_tables(x):
    return jnp.concatenate([x[:, :L], x[:, L:]], axis=0)
